# Initial kernel scaffold; baseline (speedup 1.0000x reference)
import functools

import jax
import jax.numpy as jnp
from jax import lax
from jax.experimental import pallas as pl
from jax.experimental.pallas import tpu as pltpu
from jax.experimental.pallas import tpu_sc as plsc

N = 10000
E = 320000
D_H = 64
N_GRAPHS = 64

NW = 32
NPW = 320
NPAD = NW * NPW
C = 128
EPAD = E + 2 * C
RPLEN = NPAD + 16
NEG = -1e30

_mesh = plsc.VectorSubcoreMesh(
    core_axis_name="c", subcore_axis_name="s", num_cores=2, num_subcores=16)


def _edge_kernel_body(q_hbm, s_hbm, kv_hbm, src_hbm, dst_hbm, rp_hbm, out_hbm,
                      q_v, s_v, kvc_v, rp_v, srcc_v, dstc_v, alpha_v,
                      m_v, d_v, a_v, sem):
    wid = lax.axis_index("s") * 2 + lax.axis_index("c")
    n0 = wid * NPW

    pltpu.sync_copy(q_hbm.at[pl.ds(n0, NPW)], q_v)
    pltpu.sync_copy(s_hbm.at[pl.ds(n0, NPW)], s_v)
    pltpu.sync_copy(rp_hbm.at[pl.ds(n0, NPW + 16)], rp_v)

    e0 = jnp.min(rp_v[pl.ds(0, 16)])
    e1 = jnp.min(rp_v[pl.ds(NPW, 16)])
    a0 = e0 - lax.rem(e0, 8)
    nchunks = lax.div(e1 - a0 + (C - 1), C)

    iota = lax.iota(jnp.int32, 16)

    def _init(i, _):
        a_v[pl.ds(i * 16, 16)] = jnp.zeros((16,), jnp.float32)

        @pl.when(i < NPW // 16)
        def _():
            m_v[pl.ds(i * 16, 16)] = jnp.full((16,), NEG, jnp.float32)
            d_v[pl.ds(i * 16, 16)] = jnp.zeros((16,), jnp.float32)
        return 0
    lax.fori_loop(0, (D_H * NPW) // 16, _init, 0)

    def _chunk(t, _):
        base = a0 + t * C
        pltpu.sync_copy(src_hbm.at[pl.ds(base, C)], srcc_v)
        pltpu.sync_copy(dst_hbm.at[pl.ds(base, C)], dstc_v)
        pltpu.async_copy(kv_hbm.at[srcc_v], kvc_v, sem).wait()

        dmin = jnp.full((16,), NPAD + 100, jnp.int32)
        dmax = jnp.full((16,), -1, jnp.int32)
        for g in range(C // 16):
            pos = base + g * 16 + iota
            valid = (pos >= e0) & (pos < e1)
            didx = dstc_v[pl.ds(g * 16, 16)]
            ldst = jnp.clip(didx - n0, 0, NPW - 1)
            dmin = jnp.minimum(dmin, jnp.where(valid, didx, NPAD + 100))
            dmax = jnp.maximum(dmax, jnp.where(valid, didx, -1))
            rows = iota + (g * 16)

            def _dot(d, acc):
                dfull = jnp.zeros((16,), jnp.int32) + d
                qd = plsc.load_gather(q_v, [ldst, dfull])
                kd = plsc.load_gather(kvc_v, [rows, dfull])
                return acc + qd * kd
            acc = lax.fori_loop(0, D_H, _dot, jnp.zeros((16,), jnp.float32))
            alpha_v[pl.ds(g * 16, 16)] = acc * 0.125

        dmin_s = jnp.min(dmin)
        dmax_s = jnp.max(dmax)

        g_lo = lax.div(jnp.clip(dmin_s - n0, 0, NPW - 1), 16)
        g_hi = lax.div(jnp.clip(dmax_s - n0, 0, NPW - 1), 16)

        def _grp(g, _):
            rp0 = rp_v[pl.ds(g * 16, 16)]
            rp1 = rp_v[pl.ds(g * 16 + 1, 16)]
            lo = jnp.clip(rp0 - base, 0, C)
            hi = jnp.clip(rp1 - base, 0, C)
            degc = hi - lo

            def _jmax(j, mc):
                msk = j < degc
                av = plsc.load_gather(alpha_v, [jnp.clip(lo + j, 0, C - 1)],
                                      mask=msk)
                return jnp.maximum(mc, jnp.where(msk, av, NEG))
            mc = lax.fori_loop(0, jnp.max(degc), _jmax,
                               jnp.full((16,), NEG, jnp.float32))
            mold = m_v[pl.ds(g * 16, 16)]
            mnew = jnp.maximum(mold, mc)
            m_v[pl.ds(g * 16, 16)] = mnew

            dold = d_v[pl.ds(g * 16, 16)]
            need = jnp.max(jnp.where(mnew > mold, dold, 0.0)) > 0.0

            def _rescale(_):
                scale = jnp.exp(mold - mnew)
                d_v[pl.ds(g * 16, 16)] = dold * scale

                def _rs(d, _):
                    off = d * NPW + g * 16
                    a_v[pl.ds(off, 16)] = a_v[pl.ds(off, 16)] * scale
                    return 0
                lax.fori_loop(0, D_H, _rs, 0)
                return 0
            lax.cond(need, _rescale, lambda _: 0, 0)
            return 0
        lax.fori_loop(g_lo, g_hi + 1, _grp, 0)

        for g in range(C // 16):
            pos = base + g * 16 + iota
            valid = (pos >= e0) & (pos < e1)
            didx = dstc_v[pl.ds(g * 16, 16)]
            ldst = jnp.clip(didx - n0, 0, NPW - 1)
            rows = iota + (g * 16)
            mg = plsc.load_gather(m_v, [ldst])
            ex = jnp.exp(jnp.minimum(alpha_v[pl.ds(g * 16, 16)] - mg, 80.0))
            ex = jnp.where(valid, ex, 0.0)
            plsc.addupdate_scatter(d_v, [ldst], ex)

            def _acc(d, _):
                dfull = jnp.full((16,), D_H, jnp.int32) + d
                vd = plsc.load_gather(kvc_v, [rows, dfull])
                plsc.addupdate_scatter(a_v, [d * NPW + ldst], ex * vd)
                return 0
            lax.fori_loop(0, D_H, _acc, 0)
        return 0

    lax.fori_loop(0, nchunks, _chunk, 0)

    def _fin(g, _):
        nidx = iota + g * 16
        dg = d_v[pl.ds(g * 16, 16)]
        rcp = 1.0 / (dg + 1e-16)

        def _fd(d, _):
            av = a_v[pl.ds(d * NPW + g * 16, 16)]
            dfull = jnp.zeros((16,), jnp.int32) + d
            sv = plsc.load_gather(s_v, [nidx, dfull])
            plsc.store_scatter(q_v, [nidx, dfull], av * rcp + sv)
            return 0
        lax.fori_loop(0, D_H, _fd, 0)
        return 0
    lax.fori_loop(0, NPW // 16, _fin, 0)
    pltpu.sync_copy(q_v, out_hbm.at[pl.ds(n0, NPW)])


_edge_kernel = pl.kernel(
    _edge_kernel_body,
    out_type=jax.ShapeDtypeStruct((NPAD, D_H), jnp.float32),
    mesh=_mesh,
    scratch_types=[
        pltpu.VMEM((NPW, D_H), jnp.float32),
        pltpu.VMEM((NPW, D_H), jnp.float32),
        pltpu.VMEM((C, 2 * D_H), jnp.float32),
        pltpu.VMEM((NPW + 16,), jnp.int32),
        pltpu.VMEM((C,), jnp.int32),
        pltpu.VMEM((C,), jnp.int32),
        pltpu.VMEM((C,), jnp.float32),
        pltpu.VMEM((NPW,), jnp.float32),
        pltpu.VMEM((NPW,), jnp.float32),
        pltpu.VMEM((D_H * NPW,), jnp.float32),
        pltpu.SemaphoreType.DMA,
    ],
)


def _matmul_body(x_ref, w_ref, b_ref, kv_ref, q_ref, s_ref):
    acc = jnp.dot(x_ref[...], w_ref[...],
                  preferred_element_type=jnp.float32) + b_ref[...]
    kv_ref[...] = acc[:, : 2 * D_H]
    q_ref[...] = acc[:, 2 * D_H: 3 * D_H]
    s_ref[...] = acc[:, 3 * D_H:]


def _qkvs_matmul(h, wall, ball):
    din = h.shape[1]
    bm = 1024
    grid = NPAD // bm
    return pl.pallas_call(
        _matmul_body,
        grid=(grid,),
        in_specs=[
            pl.BlockSpec((bm, din), lambda i: (i, 0)),
            pl.BlockSpec((din, 4 * D_H), lambda i: (0, 0)),
            pl.BlockSpec((1, 4 * D_H), lambda i: (0, 0)),
        ],
        out_specs=[
            pl.BlockSpec((bm, 2 * D_H), lambda i: (i, 0)),
            pl.BlockSpec((bm, D_H), lambda i: (i, 0)),
            pl.BlockSpec((bm, D_H), lambda i: (i, 0)),
        ],
        out_shape=[
            jax.ShapeDtypeStruct((NPAD, 2 * D_H), jnp.float32),
            jax.ShapeDtypeStruct((NPAD, D_H), jnp.float32),
            jax.ShapeDtypeStruct((NPAD, D_H), jnp.float32),
        ],
    )(h, wall, ball)


def _pool_body(h_ref, b_ref, wf_ref, bf_ref, o_ref, acc_ref, cnt_ref):
    i = pl.program_id(0)

    @pl.when(i == 0)
    def _():
        acc_ref[...] = jnp.zeros_like(acc_ref)
        cnt_ref[...] = jnp.zeros_like(cnt_ref)

    oh = (b_ref[...] == lax.broadcasted_iota(jnp.int32, (1, N_GRAPHS), 1)
          ).astype(jnp.float32)
    acc_ref[...] += lax.dot_general(oh, h_ref[...], (((0,), (0,)), ((), ())),
                                    preferred_element_type=jnp.float32)
    cnt_ref[...] += lax.dot_general(oh, jnp.ones_like(h_ref[..., :1]),
                                    (((0,), (0,)), ((), ())),
                                    preferred_element_type=jnp.float32)

    @pl.when(i == pl.num_programs(0) - 1)
    def _():
        pooled = acc_ref[...] / jnp.maximum(cnt_ref[...], 1.0)
        o_ref[...] = jnp.dot(pooled, wf_ref[...],
                             preferred_element_type=jnp.float32) + bf_ref[...]


def _pool(h, batch2d, wfin, bfin2d):
    bm = 1024
    return pl.pallas_call(
        _pool_body,
        grid=(NPAD // bm,),
        in_specs=[
            pl.BlockSpec((bm, D_H), lambda i: (i, 0)),
            pl.BlockSpec((bm, 1), lambda i: (i, 0)),
            pl.BlockSpec((D_H, 5), lambda i: (0, 0)),
            pl.BlockSpec((1, 5), lambda i: (0, 0)),
        ],
        out_specs=pl.BlockSpec((N_GRAPHS, 5), lambda i: (0, 0)),
        out_shape=jax.ShapeDtypeStruct((N_GRAPHS, 5), jnp.float32),
        scratch_shapes=[
            pltpu.VMEM((N_GRAPHS, N_GRAPHS), jnp.float32),
            pltpu.VMEM((N_GRAPHS, 1), jnp.float32),
        ],
    )(h, batch2d, wfin, bfin2d)


def kernel(x, edge_index, batch,
           Wq0, bq0, Wk0, bk0, Wv0, bv0, Ws0, bs0,
           Wq1, bq1, Wk1, bk1, Wv1, bv1, Ws1, bs1,
           Wq2, bq2, Wk2, bk2, Wv2, bv2, Ws2, bs2,
           Wq3, bq3, Wk3, bk3, Wv3, bv3, Ws3, bs3,
           Wfin, bfin):
    src = edge_index[0]
    dst = edge_index[1]

    key = dst * 16384 + src
    key_s = jnp.sort(key)
    dst_s = key_s >> 14
    src_s = key_s & 16383
    rowptr = jnp.searchsorted(
        dst_s, jnp.arange(RPLEN, dtype=jnp.int32), side="left"
    ).astype(jnp.int32)
    dst_sp = jnp.concatenate(
        [dst_s, jnp.full((EPAD - E,), NPAD, jnp.int32)])
    src_sp = jnp.concatenate([src_s, jnp.zeros((EPAD - E,), jnp.int32)])

    h = jnp.concatenate(
        [x, jnp.zeros((NPAD - N, x.shape[1]), jnp.float32)], axis=0)

    layers = [
        (Wq0, bq0, Wk0, bk0, Wv0, bv0, Ws0, bs0),
        (Wq1, bq1, Wk1, bk1, Wv1, bv1, Ws1, bs1),
        (Wq2, bq2, Wk2, bk2, Wv2, bv2, Ws2, bs2),
        (Wq3, bq3, Wk3, bk3, Wv3, bv3, Ws3, bs3),
    ]
    for (Wq, bq, Wk, bk, Wv, bv, Ws, bs) in layers:
        wall = jnp.concatenate([Wk, Wv, Wq, Ws], axis=1)
        ball = jnp.concatenate([bk, bv, bq, bs]).reshape(1, 4 * D_H)
        kv, q, s = _qkvs_matmul(h, wall, ball)
        h = _edge_kernel(q, s, kv, src_sp, dst_sp, rowptr)

    batch2d = jnp.concatenate(
        [batch, jnp.full((NPAD - N,), N_GRAPHS, jnp.int32)]).reshape(NPAD, 1)
    return _pool(h, batch2d, Wfin, bfin.reshape(1, 5))


# speedup vs baseline: 4.2601x; 4.2601x over previous
import functools

import jax
import jax.numpy as jnp
from jax import lax
from jax.experimental import pallas as pl
from jax.experimental.pallas import tpu as pltpu
from jax.experimental.pallas import tpu_sc as plsc

N = 10000
E = 320000
D_H = 64
N_GRAPHS = 64

NW = 32
NPW = 320
NPAD = NW * NPW
C = 128
EPAD = E + 2 * C
RPLEN = NPAD + 16
NEG = -1e30

_mesh = plsc.VectorSubcoreMesh(
    core_axis_name="c", subcore_axis_name="s", num_cores=2, num_subcores=16)


def _edge_kernel_body(q_hbm, s_hbm, kv_hbm, src_hbm, dst_hbm, rp_hbm, out_hbm,
                      q_v, s_v, kvc_v, rp_v, srcc_v, dstc_v, alpha_v,
                      m_v, d_v, a_v, sem):
    wid = lax.axis_index("s") * 2 + lax.axis_index("c")
    n0 = wid * NPW

    pltpu.sync_copy(q_hbm.at[pl.ds(n0, NPW)], q_v)
    pltpu.sync_copy(s_hbm.at[pl.ds(n0, NPW)], s_v)
    pltpu.sync_copy(rp_hbm.at[pl.ds(n0, NPW + 16)], rp_v)

    def _imin(v):
        return jnp.min(v.astype(jnp.float32)).astype(jnp.int32)

    def _imax(v):
        return jnp.max(v.astype(jnp.float32)).astype(jnp.int32)

    e0 = _imin(rp_v[pl.ds(0, 16)])
    e1 = _imin(rp_v[pl.ds(NPW, 16)])
    a0 = pl.multiple_of(e0 - lax.rem(e0, 8), 8)
    nchunks = lax.div(e1 - a0 + (C - 1), C)

    iota = lax.iota(jnp.int32, 16)

    def _init(i, _):
        for kk in range(8):
            a_v[pl.ds(i * 128 + kk * 16, 16)] = jnp.zeros((16,), jnp.float32)
        return 0
    lax.fori_loop(0, (D_H * NPW) // 128, _init, 0)
    for _g in range(NPW // 16):
        m_v[pl.ds(_g * 16, 16)] = jnp.full((16,), NEG, jnp.float32)
        d_v[pl.ds(_g * 16, 16)] = jnp.zeros((16,), jnp.float32)

    def _chunk(t, _):
        base = pl.multiple_of(a0 + t * C, 8)
        pltpu.sync_copy(src_hbm.at[pl.ds(base, C)], srcc_v)
        pltpu.sync_copy(dst_hbm.at[pl.ds(base, C)], dstc_v)
        pltpu.async_copy(kv_hbm.at[srcc_v], kvc_v, sem).wait()

        dmin = jnp.full((16,), NPAD + 100, jnp.int32)
        dmax = jnp.full((16,), -1, jnp.int32)
        for g in range(C // 16):
            pos = base + g * 16 + iota
            valid = (pos >= e0) & (pos < e1)
            didx = dstc_v[pl.ds(g * 16, 16)]
            ldst = jnp.clip(didx - n0, 0, NPW - 1)
            dmin = jnp.minimum(dmin, jnp.where(valid, didx, NPAD + 100))
            dmax = jnp.maximum(dmax, jnp.where(valid, didx, -1))
            rows = iota + (g * 16)

            def _dot(d, acc):
                dfull = jnp.zeros((16,), jnp.int32) + d
                qd = plsc.load_gather(q_v, [ldst, dfull])
                kd = plsc.load_gather(kvc_v, [rows, dfull])
                return acc + qd * kd
            acc = plsc.parallel_loop(
                0, D_H, 1, unroll=8,
                carry=jnp.zeros((16,), jnp.float32))(_dot)
            alpha_v[pl.ds(g * 16, 16)] = acc * 0.125

        dmin_s = _imin(dmin)
        dmax_s = _imax(dmax)

        g_lo = lax.div(jnp.clip(dmin_s - n0, 0, NPW - 1), 16)
        g_hi = lax.div(jnp.clip(dmax_s - n0, 0, NPW - 1), 16)

        def _grp(g, _):
            rp0 = rp_v[pl.ds(g * 16, 16)]
            rp1 = rp_v[pl.ds(g * 16 + 1, 16)]
            lo = jnp.clip(rp0 - base, 0, C)
            hi = jnp.clip(rp1 - base, 0, C)
            degc = hi - lo

            def _jmax(j, mc):
                msk = j < degc
                av = plsc.load_gather(alpha_v, [jnp.clip(lo + j, 0, C - 1)],
                                      mask=msk)
                return jnp.maximum(mc, jnp.where(msk, av, NEG))
            mc = lax.fori_loop(0, _imax(degc), _jmax,
                               jnp.full((16,), NEG, jnp.float32))
            mold = m_v[pl.ds(g * 16, 16)]
            mnew = jnp.maximum(mold, mc)
            m_v[pl.ds(g * 16, 16)] = mnew

            dold = d_v[pl.ds(g * 16, 16)]
            need = jnp.max(jnp.where(mnew > mold, dold, 0.0)) > 0.0

            def _rescale(_):
                scale = jnp.exp(mold - mnew)
                d_v[pl.ds(g * 16, 16)] = dold * scale

                def _rs(i, _):
                    for kk in range(8):
                        off = (i * 8 + kk) * NPW + g * 16
                        a_v[pl.ds(off, 16)] = a_v[pl.ds(off, 16)] * scale
                    return 0
                lax.fori_loop(0, D_H // 8, _rs, 0)
                return 0
            lax.cond(need, _rescale, lambda _: 0, 0)
            return 0
        lax.fori_loop(g_lo, g_hi + 1, _grp, 0)

        for g in range(C // 16):
            pos = base + g * 16 + iota
            valid = (pos >= e0) & (pos < e1)
            didx = dstc_v[pl.ds(g * 16, 16)]
            ldst = jnp.clip(didx - n0, 0, NPW - 1)
            rows = iota + (g * 16)
            mg = plsc.load_gather(m_v, [ldst])
            ex = jnp.exp(jnp.minimum(alpha_v[pl.ds(g * 16, 16)] - mg, 80.0))
            ex = jnp.where(valid, ex, 0.0)
            plsc.addupdate_scatter(d_v, [ldst], ex)

            def _acc(d):
                dfull = jnp.full((16,), D_H, jnp.int32) + d
                vd = plsc.load_gather(kvc_v, [rows, dfull])
                plsc.addupdate_scatter(a_v, [d * NPW + ldst], ex * vd)
            plsc.parallel_loop(0, D_H, 1, unroll=8)(_acc)
        return 0

    lax.fori_loop(0, nchunks, _chunk, 0)

    def _fin(g, _):
        nidx = iota + g * 16
        dg = d_v[pl.ds(g * 16, 16)]
        rcp = 1.0 / (dg + 1e-16)

        def _fd(i, _):
            for kk in range(8):
                d = i * 8 + kk
                av = a_v[pl.ds(d * NPW + g * 16, 16)]
                dfull = jnp.zeros((16,), jnp.int32) + d
                sv = plsc.load_gather(s_v, [nidx, dfull])
                plsc.store_scatter(q_v, [nidx, dfull], av * rcp + sv)
            return 0
        lax.fori_loop(0, D_H // 8, _fd, 0)
        return 0
    lax.fori_loop(0, NPW // 16, _fin, 0)
    pltpu.sync_copy(q_v, out_hbm.at[pl.ds(n0, NPW)])


_edge_kernel = pl.kernel(
    _edge_kernel_body,
    out_type=jax.ShapeDtypeStruct((NPAD, D_H), jnp.float32),
    mesh=_mesh,
    compiler_params=pltpu.CompilerParams(needs_layout_passes=False),
    scratch_types=[
        pltpu.VMEM((NPW, D_H), jnp.float32),
        pltpu.VMEM((NPW, D_H), jnp.float32),
        pltpu.VMEM((C, 2 * D_H), jnp.float32),
        pltpu.VMEM((NPW + 16,), jnp.int32),
        pltpu.VMEM((C,), jnp.int32),
        pltpu.VMEM((C,), jnp.int32),
        pltpu.VMEM((C,), jnp.float32),
        pltpu.VMEM((NPW,), jnp.float32),
        pltpu.VMEM((NPW,), jnp.float32),
        pltpu.VMEM((D_H * NPW,), jnp.float32),
        pltpu.SemaphoreType.DMA,
    ],
)


def _matmul_body(x_ref, w_ref, b_ref, kv_ref, q_ref, s_ref):
    acc = jnp.dot(x_ref[...], w_ref[...],
                  preferred_element_type=jnp.float32) + b_ref[...]
    kv_ref[...] = acc[:, : 2 * D_H]
    q_ref[...] = acc[:, 2 * D_H: 3 * D_H]
    s_ref[...] = acc[:, 3 * D_H:]


def _qkvs_matmul(h, wall, ball):
    din = h.shape[1]
    bm = 1024
    grid = NPAD // bm
    return pl.pallas_call(
        _matmul_body,
        grid=(grid,),
        in_specs=[
            pl.BlockSpec((bm, din), lambda i: (i, 0)),
            pl.BlockSpec((din, 4 * D_H), lambda i: (0, 0)),
            pl.BlockSpec((1, 4 * D_H), lambda i: (0, 0)),
        ],
        out_specs=[
            pl.BlockSpec((bm, 2 * D_H), lambda i: (i, 0)),
            pl.BlockSpec((bm, D_H), lambda i: (i, 0)),
            pl.BlockSpec((bm, D_H), lambda i: (i, 0)),
        ],
        out_shape=[
            jax.ShapeDtypeStruct((NPAD, 2 * D_H), jnp.float32),
            jax.ShapeDtypeStruct((NPAD, D_H), jnp.float32),
            jax.ShapeDtypeStruct((NPAD, D_H), jnp.float32),
        ],
    )(h, wall, ball)


def _pool_body(h_ref, b_ref, wf_ref, bf_ref, o_ref, acc_ref, cnt_ref):
    i = pl.program_id(0)

    @pl.when(i == 0)
    def _():
        acc_ref[...] = jnp.zeros_like(acc_ref)
        cnt_ref[...] = jnp.zeros_like(cnt_ref)

    oh = (b_ref[...] == lax.broadcasted_iota(jnp.int32, (1, N_GRAPHS), 1)
          ).astype(jnp.float32)
    acc_ref[...] += lax.dot_general(oh, h_ref[...], (((0,), (0,)), ((), ())),
                                    preferred_element_type=jnp.float32)
    cnt_ref[...] += lax.dot_general(oh, jnp.ones_like(h_ref[..., :1]),
                                    (((0,), (0,)), ((), ())),
                                    preferred_element_type=jnp.float32)

    @pl.when(i == pl.num_programs(0) - 1)
    def _():
        pooled = acc_ref[...] / jnp.maximum(cnt_ref[...], 1.0)
        o_ref[...] = jnp.dot(pooled, wf_ref[...],
                             preferred_element_type=jnp.float32) + bf_ref[...]


def _pool(h, batch2d, wfin, bfin2d):
    bm = 1024
    return pl.pallas_call(
        _pool_body,
        grid=(NPAD // bm,),
        in_specs=[
            pl.BlockSpec((bm, D_H), lambda i: (i, 0)),
            pl.BlockSpec((bm, 1), lambda i: (i, 0)),
            pl.BlockSpec((D_H, 5), lambda i: (0, 0)),
            pl.BlockSpec((1, 5), lambda i: (0, 0)),
        ],
        out_specs=pl.BlockSpec((N_GRAPHS, 5), lambda i: (0, 0)),
        out_shape=jax.ShapeDtypeStruct((N_GRAPHS, 5), jnp.float32),
        scratch_shapes=[
            pltpu.VMEM((N_GRAPHS, N_GRAPHS), jnp.float32),
            pltpu.VMEM((N_GRAPHS, 1), jnp.float32),
        ],
    )(h, batch2d, wfin, bfin2d)


def kernel(x, edge_index, batch,
           Wq0, bq0, Wk0, bk0, Wv0, bv0, Ws0, bs0,
           Wq1, bq1, Wk1, bk1, Wv1, bv1, Ws1, bs1,
           Wq2, bq2, Wk2, bk2, Wv2, bv2, Ws2, bs2,
           Wq3, bq3, Wk3, bk3, Wv3, bv3, Ws3, bs3,
           Wfin, bfin):
    src = edge_index[0]
    dst = edge_index[1]

    key = dst * 16384 + src
    key_s = jnp.sort(key)
    dst_s = key_s >> 14
    src_s = key_s & 16383
    rowptr = jnp.searchsorted(
        dst_s, jnp.arange(RPLEN, dtype=jnp.int32), side="left"
    ).astype(jnp.int32)
    dst_sp = jnp.concatenate(
        [dst_s, jnp.full((EPAD - E,), NPAD, jnp.int32)])
    src_sp = jnp.concatenate([src_s, jnp.zeros((EPAD - E,), jnp.int32)])

    h = jnp.concatenate(
        [x, jnp.zeros((NPAD - N, x.shape[1]), jnp.float32)], axis=0)

    layers = [
        (Wq0, bq0, Wk0, bk0, Wv0, bv0, Ws0, bs0),
        (Wq1, bq1, Wk1, bk1, Wv1, bv1, Ws1, bs1),
        (Wq2, bq2, Wk2, bk2, Wv2, bv2, Ws2, bs2),
        (Wq3, bq3, Wk3, bk3, Wv3, bv3, Ws3, bs3),
    ]
    for (Wq, bq, Wk, bk, Wv, bv, Ws, bs) in layers:
        wall = jnp.concatenate([Wk, Wv, Wq, Ws], axis=1)
        ball = jnp.concatenate([bk, bv, bq, bs]).reshape(1, 4 * D_H)
        kv, q, s = _qkvs_matmul(h, wall, ball)
        h = _edge_kernel(q, s, kv, src_sp, dst_sp, rowptr)

    batch2d = jnp.concatenate(
        [batch, jnp.full((NPAD - N,), N_GRAPHS, jnp.int32)]).reshape(NPAD, 1)
    return _pool(h, batch2d, Wfin, bfin.reshape(1, 5))


# speedup vs baseline: 6.5893x; 1.5468x over previous
import jax
import jax.numpy as jnp
from jax import lax
from jax.experimental import pallas as pl
from jax.experimental.pallas import tpu as pltpu
from jax.experimental.pallas import tpu_sc as plsc

N = 10000
E = 320000
D_H = 64
N_GRAPHS = 64

NW = 32
NPW = 320
NPAD = NW * NPW
C = 128
QS = NPW + 1
CS = C + 1
EPAD = E + 2 * C
RPLEN = NPAD + 16
NEG = -1e30

_mesh = plsc.VectorSubcoreMesh(
    core_axis_name="c", subcore_axis_name="s", num_cores=2, num_subcores=16)


def _edge_kernel_body(q_hbm, s_hbm, kv_hbm, src_hbm, dst_hbm, rp_hbm, out_hbm,
                      qs_v, s_v, qt_v, kvc_v, kvt_v, rp_v, srcc_v, dstc_v,
                      alpha_v, m_v, d_v, a_v, sem):
    wid = lax.axis_index("s") * 2 + lax.axis_index("c")
    n0 = wid * NPW

    pltpu.sync_copy(q_hbm.at[pl.ds(n0 * D_H, NPW * D_H)], qs_v)
    pltpu.sync_copy(s_hbm.at[pl.ds(n0 * D_H, NPW * D_H)], s_v)
    pltpu.sync_copy(rp_hbm.at[pl.ds(n0, NPW + 16)], rp_v)

    def _imin(v):
        return jnp.min(v.astype(jnp.float32)).astype(jnp.int32)

    def _imax(v):
        return jnp.max(v.astype(jnp.float32)).astype(jnp.int32)

    e0 = _imin(rp_v[pl.ds(0, 16)])
    e1 = _imin(rp_v[pl.ds(NPW, 16)])
    a0 = pl.multiple_of(e0 - lax.rem(e0, 8), 8)
    nchunks = lax.div(e1 - a0 + (C - 1), C)

    iota = lax.iota(jnp.int32, 16)
    iota_qs = iota * QS
    iota_cs = iota * CS

    def _qt(n):
        for k in range(4):
            row = qs_v[pl.ds(n * D_H + k * 16, 16)]
            plsc.store_scatter(qt_v, [iota_qs + (k * 16 * QS) + n], row)
    plsc.parallel_loop(0, NPW, 1, unroll=4)(_qt)

    def _init(i, _):
        for kk in range(8):
            a_v[pl.ds(i * 128 + kk * 16, 16)] = jnp.zeros((16,), jnp.float32)
        return 0
    lax.fori_loop(0, (D_H * NPW) // 128, _init, 0)
    for _g in range(NPW // 16):
        m_v[pl.ds(_g * 16, 16)] = jnp.full((16,), NEG, jnp.float32)
        d_v[pl.ds(_g * 16, 16)] = jnp.zeros((16,), jnp.float32)

    def _chunk(t, _):
        base = pl.multiple_of(a0 + t * C, 8)
        pltpu.sync_copy(src_hbm.at[pl.ds(base, C)], srcc_v)
        pltpu.sync_copy(dst_hbm.at[pl.ds(base, C)], dstc_v)
        pltpu.async_copy(kv_hbm.at[srcc_v], kvc_v, sem).wait()

        def _tr(e):
            efull = jnp.zeros((16,), jnp.int32) + e
            for k in range(8):
                cvec = plsc.load_gather(kvc_v, [efull, iota + (k * 16)])
                plsc.store_scatter(kvt_v, [iota_cs + (k * 16 * CS) + e], cvec)
        plsc.parallel_loop(0, C, 1, unroll=4)(_tr)

        dmin = jnp.full((16,), NPAD + 100, jnp.int32)
        dmax = jnp.full((16,), -1, jnp.int32)
        for g in range(C // 16):
            pos = base + g * 16 + iota
            valid = (pos >= e0) & (pos < e1)
            didx = dstc_v[pl.ds(g * 16, 16)]
            ldst = jnp.clip(didx - n0, 0, NPW - 1)
            dmin = jnp.minimum(dmin, jnp.where(valid, didx, NPAD + 100))
            dmax = jnp.maximum(dmax, jnp.where(valid, didx, -1))

            def _dot(d, acc):
                kt = kvt_v[pl.ds(d * CS + g * 16, 16)]
                qd = plsc.load_gather(qt_v, [ldst + d * QS])
                return acc + kt * qd
            acc = plsc.parallel_loop(
                0, D_H, 1, unroll=8,
                carry=jnp.zeros((16,), jnp.float32))(_dot)
            alpha_v[pl.ds(g * 16, 16)] = acc * 0.125

        dmin_s = _imin(dmin)
        dmax_s = _imax(dmax)

        g_lo = lax.div(jnp.clip(dmin_s - n0, 0, NPW - 1), 16)
        g_hi = lax.div(jnp.clip(dmax_s - n0, 0, NPW - 1), 16)

        def _grp(g, _):
            rp0 = rp_v[pl.ds(g * 16, 16)]
            rp1 = rp_v[pl.ds(g * 16 + 1, 16)]
            lo = jnp.clip(rp0 - base, 0, C)
            hi = jnp.clip(rp1 - base, 0, C)
            degc = hi - lo

            def _jmax(j, mc):
                msk = j < degc
                av = plsc.load_gather(alpha_v, [jnp.clip(lo + j, 0, C - 1)],
                                      mask=msk)
                return jnp.maximum(mc, jnp.where(msk, av, NEG))
            mc = lax.fori_loop(0, _imax(degc), _jmax,
                               jnp.full((16,), NEG, jnp.float32))
            mold = m_v[pl.ds(g * 16, 16)]
            mnew = jnp.maximum(mold, mc)
            m_v[pl.ds(g * 16, 16)] = mnew

            dold = d_v[pl.ds(g * 16, 16)]
            need = jnp.max(jnp.where(mnew > mold, dold, 0.0)) > 0.0

            def _rescale(_):
                scale = jnp.exp(mold - mnew)
                d_v[pl.ds(g * 16, 16)] = dold * scale

                def _rs(j, _):
                    nloc = g * 16 + j
                    sc = jnp.take(scale, jnp.zeros((16,), jnp.int32) + j)
                    for k in range(4):
                        off = nloc * D_H + k * 16
                        a_v[pl.ds(off, 16)] = a_v[pl.ds(off, 16)] * sc
                    return 0
                lax.fori_loop(0, 16, _rs, 0)
                return 0
            lax.cond(need, _rescale, lambda _: 0, 0)
            return 0
        lax.fori_loop(g_lo, g_hi + 1, _grp, 0)

        for g in range(C // 16):
            pos = base + g * 16 + iota
            valid = (pos >= e0) & (pos < e1)
            didx = dstc_v[pl.ds(g * 16, 16)]
            ldst = jnp.clip(didx - n0, 0, NPW - 1)
            mg = plsc.load_gather(m_v, [ldst])
            ex = jnp.exp(jnp.minimum(alpha_v[pl.ds(g * 16, 16)] - mg, 80.0))
            ex = jnp.where(valid, ex, 0.0)
            plsc.addupdate_scatter(d_v, [ldst], ex)
            ldst64 = ldst * D_H

            def _acc(d):
                vt = kvt_v[pl.ds((D_H + d) * CS + g * 16, 16)]
                plsc.addupdate_scatter(a_v, [ldst64 + d], ex * vt)
            plsc.parallel_loop(0, D_H, 1, unroll=8)(_acc)
        return 0

    lax.fori_loop(0, nchunks, _chunk, 0)

    def _fin(n):
        dv = d_v[pl.ds(jnp.minimum(n, NPW - 16), 16)]
        shift = n - jnp.minimum(n, NPW - 16)
        rcpv = 1.0 / (dv + 1e-16)
        rcp = jnp.take(rcpv, jnp.zeros((16,), jnp.int32) + shift)
        for k in range(4):
            off = n * D_H + k * 16
            av = a_v[pl.ds(off, 16)]
            sv = s_v[pl.ds(off, 16)]
            qs_v[pl.ds(off, 16)] = av * rcp + sv
    plsc.parallel_loop(0, NPW, 1, unroll=2)(_fin)
    pltpu.sync_copy(qs_v, out_hbm.at[pl.ds(n0 * D_H, NPW * D_H)])


_edge_kernel = pl.kernel(
    _edge_kernel_body,
    out_type=jax.ShapeDtypeStruct((NPAD * D_H,), jnp.float32),
    mesh=_mesh,
    compiler_params=pltpu.CompilerParams(needs_layout_passes=False),
    scratch_types=[
        pltpu.VMEM((NPW * D_H,), jnp.float32),
        pltpu.VMEM((NPW * D_H,), jnp.float32),
        pltpu.VMEM((D_H * QS,), jnp.float32),
        pltpu.VMEM((C, 2 * D_H), jnp.float32),
        pltpu.VMEM((2 * D_H * CS,), jnp.float32),
        pltpu.VMEM((NPW + 16,), jnp.int32),
        pltpu.VMEM((C,), jnp.int32),
        pltpu.VMEM((C,), jnp.int32),
        pltpu.VMEM((C,), jnp.float32),
        pltpu.VMEM((NPW,), jnp.float32),
        pltpu.VMEM((NPW + 16,), jnp.float32),
        pltpu.VMEM((NPW * D_H,), jnp.float32),
        pltpu.SemaphoreType.DMA,
    ],
)


def _matmul_body(x_ref, w_ref, b_ref, kv_ref, q_ref, s_ref):
    acc = jnp.dot(x_ref[...], w_ref[...],
                  preferred_element_type=jnp.float32) + b_ref[...]
    kv_ref[...] = acc[:, : 2 * D_H]
    q_ref[...] = acc[:, 2 * D_H: 3 * D_H]
    s_ref[...] = acc[:, 3 * D_H:]


def _qkvs_matmul(h, wall, ball):
    din = h.shape[1]
    bm = 1024
    grid = NPAD // bm
    return pl.pallas_call(
        _matmul_body,
        grid=(grid,),
        in_specs=[
            pl.BlockSpec((bm, din), lambda i: (i, 0)),
            pl.BlockSpec((din, 4 * D_H), lambda i: (0, 0)),
            pl.BlockSpec((1, 4 * D_H), lambda i: (0, 0)),
        ],
        out_specs=[
            pl.BlockSpec((bm, 2 * D_H), lambda i: (i, 0)),
            pl.BlockSpec((bm, D_H), lambda i: (i, 0)),
            pl.BlockSpec((bm, D_H), lambda i: (i, 0)),
        ],
        out_shape=[
            jax.ShapeDtypeStruct((NPAD, 2 * D_H), jnp.float32),
            jax.ShapeDtypeStruct((NPAD, D_H), jnp.float32),
            jax.ShapeDtypeStruct((NPAD, D_H), jnp.float32),
        ],
    )(h, wall, ball)


def _pool_body(h_ref, b_ref, wf_ref, bf_ref, o_ref, acc_ref, cnt_ref):
    i = pl.program_id(0)

    @pl.when(i == 0)
    def _():
        acc_ref[...] = jnp.zeros_like(acc_ref)
        cnt_ref[...] = jnp.zeros_like(cnt_ref)

    oh = (b_ref[...] == lax.broadcasted_iota(jnp.int32, (1, N_GRAPHS), 1)
          ).astype(jnp.float32)
    acc_ref[...] += lax.dot_general(oh, h_ref[...], (((0,), (0,)), ((), ())),
                                    preferred_element_type=jnp.float32)
    cnt_ref[...] += lax.dot_general(oh, jnp.ones_like(h_ref[..., :1]),
                                    (((0,), (0,)), ((), ())),
                                    preferred_element_type=jnp.float32)

    @pl.when(i == pl.num_programs(0) - 1)
    def _():
        pooled = acc_ref[...] / jnp.maximum(cnt_ref[...], 1.0)
        o_ref[...] = jnp.dot(pooled, wf_ref[...],
                             preferred_element_type=jnp.float32) + bf_ref[...]


def _pool(h, batch2d, wfin, bfin2d):
    bm = 1024
    return pl.pallas_call(
        _pool_body,
        grid=(NPAD // bm,),
        in_specs=[
            pl.BlockSpec((bm, D_H), lambda i: (i, 0)),
            pl.BlockSpec((bm, 1), lambda i: (i, 0)),
            pl.BlockSpec((D_H, 5), lambda i: (0, 0)),
            pl.BlockSpec((1, 5), lambda i: (0, 0)),
        ],
        out_specs=pl.BlockSpec((N_GRAPHS, 5), lambda i: (0, 0)),
        out_shape=jax.ShapeDtypeStruct((N_GRAPHS, 5), jnp.float32),
        scratch_shapes=[
            pltpu.VMEM((N_GRAPHS, N_GRAPHS), jnp.float32),
            pltpu.VMEM((N_GRAPHS, 1), jnp.float32),
        ],
    )(h, batch2d, wfin, bfin2d)


def kernel(x, edge_index, batch,
           Wq0, bq0, Wk0, bk0, Wv0, bv0, Ws0, bs0,
           Wq1, bq1, Wk1, bk1, Wv1, bv1, Ws1, bs1,
           Wq2, bq2, Wk2, bk2, Wv2, bv2, Ws2, bs2,
           Wq3, bq3, Wk3, bk3, Wv3, bv3, Ws3, bs3,
           Wfin, bfin):
    src = edge_index[0]
    dst = edge_index[1]

    key = dst * 16384 + src
    key_s = jnp.sort(key)
    dst_s = key_s >> 14
    src_s = key_s & 16383
    rowptr = jnp.searchsorted(
        dst_s, jnp.arange(RPLEN, dtype=jnp.int32), side="left"
    ).astype(jnp.int32)
    dst_sp = jnp.concatenate(
        [dst_s, jnp.full((EPAD - E,), NPAD, jnp.int32)])
    src_sp = jnp.concatenate([src_s, jnp.zeros((EPAD - E,), jnp.int32)])

    h = jnp.concatenate(
        [x, jnp.zeros((NPAD - N, x.shape[1]), jnp.float32)], axis=0)

    layers = [
        (Wq0, bq0, Wk0, bk0, Wv0, bv0, Ws0, bs0),
        (Wq1, bq1, Wk1, bk1, Wv1, bv1, Ws1, bs1),
        (Wq2, bq2, Wk2, bk2, Wv2, bv2, Ws2, bs2),
        (Wq3, bq3, Wk3, bk3, Wv3, bv3, Ws3, bs3),
    ]
    for (Wq, bq, Wk, bk, Wv, bv, Ws, bs) in layers:
        wall = jnp.concatenate([Wk, Wv, Wq, Ws], axis=1)
        ball = jnp.concatenate([bk, bv, bq, bs]).reshape(1, 4 * D_H)
        kv, q, s = _qkvs_matmul(h, wall, ball)
        hflat = _edge_kernel(q.reshape(-1), s.reshape(-1), kv,
                             src_sp, dst_sp, rowptr)
        h = hflat.reshape(NPAD, D_H)

    batch2d = jnp.concatenate(
        [batch, jnp.full((NPAD - N,), N_GRAPHS, jnp.int32)]).reshape(NPAD, 1)
    return _pool(h, batch2d, Wfin, bfin.reshape(1, 5))


# speedup vs baseline: 7.6318x; 1.1582x over previous
import jax
import jax.numpy as jnp
from jax import lax
from jax.experimental import pallas as pl
from jax.experimental.pallas import tpu as pltpu
from jax.experimental.pallas import tpu_sc as plsc

N = 10000
E = 320000
D_H = 64
N_GRAPHS = 64

NW = 32
NPW = 320
NPAD = NW * NPW
C = 128
QS = NPW + 1
CS = C + 1
EPAD = E + 2 * C
RPLEN = NPAD + 16
NEG = -1e30

_mesh = plsc.VectorSubcoreMesh(
    core_axis_name="c", subcore_axis_name="s", num_cores=2, num_subcores=16)


def _edge_kernel_body(q_hbm, s_hbm, kv_hbm, src_hbm, dst_hbm, rp_hbm, out_hbm,
                      qs_v, s_v, qt_v, kvc_v, kvt_v, rp_v, srcc_v, dstc_v,
                      alpha_v, m_v, d_v, a_v, sem, sem_sd):
    wid = lax.axis_index("s") * 2 + lax.axis_index("c")
    n0 = wid * NPW

    pltpu.sync_copy(q_hbm.at[pl.ds(n0 * D_H, NPW * D_H)], qs_v)
    pltpu.sync_copy(s_hbm.at[pl.ds(n0 * D_H, NPW * D_H)], s_v)
    pltpu.sync_copy(rp_hbm.at[pl.ds(n0, NPW + 16)], rp_v)

    def _imin(v):
        return jnp.min(v.astype(jnp.float32)).astype(jnp.int32)

    def _imax(v):
        return jnp.max(v.astype(jnp.float32)).astype(jnp.int32)

    e0 = _imin(rp_v[pl.ds(0, 16)])
    e1 = _imin(rp_v[pl.ds(NPW, 16)])
    a0 = pl.multiple_of(e0 - lax.rem(e0, 8), 8)
    nchunks = lax.div(e1 - a0 + (C - 1), C)

    iota = lax.iota(jnp.int32, 16)
    iota_qs = iota * QS
    iota_cs = iota * CS

    def _qt(n):
        for k in range(4):
            row = qs_v[pl.ds(n * D_H + k * 16, 16)]
            plsc.store_scatter(qt_v, [iota_qs + (k * 16 * QS) + n], row)
    plsc.parallel_loop(0, NPW, 1, unroll=4)(_qt)

    def _init(i, _):
        for kk in range(8):
            a_v[pl.ds(i * 128 + kk * 16, 16)] = jnp.zeros((16,), jnp.float32)
        return 0
    lax.fori_loop(0, (D_H * NPW) // 128, _init, 0)
    for _g in range(NPW // 16):
        m_v[pl.ds(_g * 16, 16)] = jnp.full((16,), NEG, jnp.float32)
        d_v[pl.ds(_g * 16, 16)] = jnp.zeros((16,), jnp.float32)

    @pl.when(nchunks > 0)
    def _():
        b0 = pl.multiple_of(a0, 8)
        pltpu.sync_copy(src_hbm.at[pl.ds(b0, C)], srcc_v.at[pl.ds(0, C)])
        pltpu.sync_copy(dst_hbm.at[pl.ds(b0, C)], dstc_v.at[pl.ds(0, C)])
        pltpu.async_copy(kv_hbm.at[srcc_v.at[pl.ds(0, C)]], kvc_v, sem)

        @pl.when(nchunks > 1)
        def _():
            b1 = pl.multiple_of(a0 + C, 8)
            pltpu.async_copy(src_hbm.at[pl.ds(b1, C)],
                             srcc_v.at[pl.ds(C, C)], sem_sd)
            pltpu.async_copy(dst_hbm.at[pl.ds(b1, C)],
                             dstc_v.at[pl.ds(C, C)], sem_sd)

    def _chunk(t, _):
        base = pl.multiple_of(a0 + t * C, 8)
        boff = lax.rem(t, 2) * C
        pltpu.make_async_copy(kv_hbm.at[srcc_v.at[pl.ds(boff, C)]],
                              kvc_v, sem).wait()

        def _tr(e):
            efull = jnp.zeros((16,), jnp.int32) + e
            for k in range(8):
                cvec = plsc.load_gather(kvc_v, [efull, iota + (k * 16)])
                plsc.store_scatter(kvt_v, [iota_cs + (k * 16 * CS) + e], cvec)
        plsc.parallel_loop(0, C, 1, unroll=4)(_tr)

        @pl.when(t + 1 < nchunks)
        def _():
            b1off = lax.rem(t + 1, 2) * C
            base1 = pl.multiple_of(a0 + (t + 1) * C, 8)
            pltpu.make_async_copy(src_hbm.at[pl.ds(base1, C)],
                                  srcc_v.at[pl.ds(b1off, C)], sem_sd).wait()
            pltpu.make_async_copy(dst_hbm.at[pl.ds(base1, C)],
                                  dstc_v.at[pl.ds(b1off, C)], sem_sd).wait()
            pltpu.async_copy(kv_hbm.at[srcc_v.at[pl.ds(b1off, C)]],
                             kvc_v, sem)

        dmin = jnp.full((16,), NPAD + 100, jnp.int32)
        dmax = jnp.full((16,), -1, jnp.int32)
        for g in range(C // 16):
            pos = base + g * 16 + iota
            valid = (pos >= e0) & (pos < e1)
            didx = dstc_v[pl.ds(boff + g * 16, 16)]
            ldst = jnp.clip(didx - n0, 0, NPW - 1)
            dmin = jnp.minimum(dmin, jnp.where(valid, didx, NPAD + 100))
            dmax = jnp.maximum(dmax, jnp.where(valid, didx, -1))

            def _dot(d, acc):
                kt = kvt_v[pl.ds(d * CS + g * 16, 16)]
                qd = plsc.load_gather(qt_v, [ldst + d * QS])
                return acc + kt * qd
            acc = plsc.parallel_loop(
                0, D_H, 1, unroll=8,
                carry=jnp.zeros((16,), jnp.float32))(_dot)
            alpha_v[pl.ds(g * 16, 16)] = acc * 0.125

        dmin_s = _imin(dmin)
        dmax_s = _imax(dmax)

        g_lo = lax.div(jnp.clip(dmin_s - n0, 0, NPW - 1), 16)
        g_hi = lax.div(jnp.clip(dmax_s - n0, 0, NPW - 1), 16)

        def _grp(g, _):
            rp0 = rp_v[pl.ds(g * 16, 16)]
            rp1 = rp_v[pl.ds(g * 16 + 1, 16)]
            lo = jnp.clip(rp0 - base, 0, C)
            hi = jnp.clip(rp1 - base, 0, C)
            degc = hi - lo

            def _jmax(j, mc):
                msk = j < degc
                av = plsc.load_gather(alpha_v, [jnp.clip(lo + j, 0, C - 1)],
                                      mask=msk)
                return jnp.maximum(mc, jnp.where(msk, av, NEG))
            mc = lax.fori_loop(0, _imax(degc), _jmax,
                               jnp.full((16,), NEG, jnp.float32))
            mold = m_v[pl.ds(g * 16, 16)]
            mnew = jnp.maximum(mold, mc)
            m_v[pl.ds(g * 16, 16)] = mnew

            dold = d_v[pl.ds(g * 16, 16)]
            need = jnp.max(jnp.where(mnew > mold, dold, 0.0)) > 0.0

            def _rescale(_):
                scale = jnp.exp(mold - mnew)
                d_v[pl.ds(g * 16, 16)] = dold * scale

                def _rs(j, _):
                    nloc = g * 16 + j
                    sc = jnp.take(scale, jnp.zeros((16,), jnp.int32) + j)
                    for k in range(4):
                        off = nloc * D_H + k * 16
                        a_v[pl.ds(off, 16)] = a_v[pl.ds(off, 16)] * sc
                    return 0
                lax.fori_loop(0, 16, _rs, 0)
                return 0
            lax.cond(need, _rescale, lambda _: 0, 0)
            return 0
        lax.fori_loop(g_lo, g_hi + 1, _grp, 0)

        for g in range(C // 16):
            pos = base + g * 16 + iota
            valid = (pos >= e0) & (pos < e1)
            didx = dstc_v[pl.ds(boff + g * 16, 16)]
            ldst = jnp.clip(didx - n0, 0, NPW - 1)
            mg = plsc.load_gather(m_v, [ldst])
            ex = jnp.exp(jnp.minimum(alpha_v[pl.ds(g * 16, 16)] - mg, 80.0))
            ex = jnp.where(valid, ex, 0.0)
            plsc.addupdate_scatter(d_v, [ldst], ex)
            ldst64 = ldst * D_H

            def _acc(d):
                vt = kvt_v[pl.ds((D_H + d) * CS + g * 16, 16)]
                plsc.addupdate_scatter(a_v, [ldst64 + d], ex * vt)
            plsc.parallel_loop(0, D_H, 1, unroll=8)(_acc)

        @pl.when(t + 2 < nchunks)
        def _():
            b2off = lax.rem(t, 2) * C
            base2 = pl.multiple_of(a0 + (t + 2) * C, 8)
            pltpu.async_copy(src_hbm.at[pl.ds(base2, C)],
                             srcc_v.at[pl.ds(b2off, C)], sem_sd)
            pltpu.async_copy(dst_hbm.at[pl.ds(base2, C)],
                             dstc_v.at[pl.ds(b2off, C)], sem_sd)
        return 0

    lax.fori_loop(0, nchunks, _chunk, 0)

    def _fin(n):
        dv = d_v[pl.ds(jnp.minimum(n, NPW - 16), 16)]
        shift = n - jnp.minimum(n, NPW - 16)
        rcpv = 1.0 / (dv + 1e-16)
        rcp = jnp.take(rcpv, jnp.zeros((16,), jnp.int32) + shift)
        for k in range(4):
            off = n * D_H + k * 16
            av = a_v[pl.ds(off, 16)]
            sv = s_v[pl.ds(off, 16)]
            qs_v[pl.ds(off, 16)] = av * rcp + sv
    plsc.parallel_loop(0, NPW, 1, unroll=2)(_fin)
    pltpu.sync_copy(qs_v, out_hbm.at[pl.ds(n0 * D_H, NPW * D_H)])


_edge_kernel = pl.kernel(
    _edge_kernel_body,
    out_type=jax.ShapeDtypeStruct((NPAD * D_H,), jnp.float32),
    mesh=_mesh,
    compiler_params=pltpu.CompilerParams(needs_layout_passes=False),
    scratch_types=[
        pltpu.VMEM((NPW * D_H,), jnp.float32),
        pltpu.VMEM((NPW * D_H,), jnp.float32),
        pltpu.VMEM((D_H * QS,), jnp.float32),
        pltpu.VMEM((C, 2 * D_H), jnp.float32),
        pltpu.VMEM((2 * D_H * CS,), jnp.float32),
        pltpu.VMEM((NPW + 16,), jnp.int32),
        pltpu.VMEM((2 * C,), jnp.int32),
        pltpu.VMEM((2 * C,), jnp.int32),
        pltpu.VMEM((C,), jnp.float32),
        pltpu.VMEM((NPW,), jnp.float32),
        pltpu.VMEM((NPW + 16,), jnp.float32),
        pltpu.VMEM((NPW * D_H,), jnp.float32),
        pltpu.SemaphoreType.DMA,
        pltpu.SemaphoreType.DMA,
    ],
)


def _matmul_body(x_ref, w_ref, b_ref, kv_ref, q_ref, s_ref):
    acc = jnp.dot(x_ref[...], w_ref[...],
                  preferred_element_type=jnp.float32) + b_ref[...]
    kv_ref[...] = acc[:, : 2 * D_H]
    q_ref[...] = acc[:, 2 * D_H: 3 * D_H]
    s_ref[...] = acc[:, 3 * D_H:]


def _qkvs_matmul(h, wall, ball):
    din = h.shape[1]
    bm = 1024
    grid = NPAD // bm
    return pl.pallas_call(
        _matmul_body,
        grid=(grid,),
        in_specs=[
            pl.BlockSpec((bm, din), lambda i: (i, 0)),
            pl.BlockSpec((din, 4 * D_H), lambda i: (0, 0)),
            pl.BlockSpec((1, 4 * D_H), lambda i: (0, 0)),
        ],
        out_specs=[
            pl.BlockSpec((bm, 2 * D_H), lambda i: (i, 0)),
            pl.BlockSpec((bm, D_H), lambda i: (i, 0)),
            pl.BlockSpec((bm, D_H), lambda i: (i, 0)),
        ],
        out_shape=[
            jax.ShapeDtypeStruct((NPAD, 2 * D_H), jnp.float32),
            jax.ShapeDtypeStruct((NPAD, D_H), jnp.float32),
            jax.ShapeDtypeStruct((NPAD, D_H), jnp.float32),
        ],
    )(h, wall, ball)


def _pool_body(h_ref, b_ref, wf_ref, bf_ref, o_ref, acc_ref, cnt_ref):
    i = pl.program_id(0)

    @pl.when(i == 0)
    def _():
        acc_ref[...] = jnp.zeros_like(acc_ref)
        cnt_ref[...] = jnp.zeros_like(cnt_ref)

    oh = (b_ref[...] == lax.broadcasted_iota(jnp.int32, (1, N_GRAPHS), 1)
          ).astype(jnp.float32)
    acc_ref[...] += lax.dot_general(oh, h_ref[...], (((0,), (0,)), ((), ())),
                                    preferred_element_type=jnp.float32)
    cnt_ref[...] += lax.dot_general(oh, jnp.ones_like(h_ref[..., :1]),
                                    (((0,), (0,)), ((), ())),
                                    preferred_element_type=jnp.float32)

    @pl.when(i == pl.num_programs(0) - 1)
    def _():
        pooled = acc_ref[...] / jnp.maximum(cnt_ref[...], 1.0)
        o_ref[...] = jnp.dot(pooled, wf_ref[...],
                             preferred_element_type=jnp.float32) + bf_ref[...]


def _pool(h, batch2d, wfin, bfin2d):
    bm = 1024
    return pl.pallas_call(
        _pool_body,
        grid=(NPAD // bm,),
        in_specs=[
            pl.BlockSpec((bm, D_H), lambda i: (i, 0)),
            pl.BlockSpec((bm, 1), lambda i: (i, 0)),
            pl.BlockSpec((D_H, 5), lambda i: (0, 0)),
            pl.BlockSpec((1, 5), lambda i: (0, 0)),
        ],
        out_specs=pl.BlockSpec((N_GRAPHS, 5), lambda i: (0, 0)),
        out_shape=jax.ShapeDtypeStruct((N_GRAPHS, 5), jnp.float32),
        scratch_shapes=[
            pltpu.VMEM((N_GRAPHS, N_GRAPHS), jnp.float32),
            pltpu.VMEM((N_GRAPHS, 1), jnp.float32),
        ],
    )(h, batch2d, wfin, bfin2d)


def kernel(x, edge_index, batch,
           Wq0, bq0, Wk0, bk0, Wv0, bv0, Ws0, bs0,
           Wq1, bq1, Wk1, bk1, Wv1, bv1, Ws1, bs1,
           Wq2, bq2, Wk2, bk2, Wv2, bv2, Ws2, bs2,
           Wq3, bq3, Wk3, bk3, Wv3, bv3, Ws3, bs3,
           Wfin, bfin):
    src = edge_index[0]
    dst = edge_index[1]

    key = dst * 16384 + src
    key_s = jnp.sort(key)
    dst_s = key_s >> 14
    src_s = key_s & 16383
    rowptr = jnp.searchsorted(
        dst_s, jnp.arange(RPLEN, dtype=jnp.int32), side="left"
    ).astype(jnp.int32)
    dst_sp = jnp.concatenate(
        [dst_s, jnp.full((EPAD - E,), NPAD, jnp.int32)])
    src_sp = jnp.concatenate([src_s, jnp.zeros((EPAD - E,), jnp.int32)])

    h = jnp.concatenate(
        [x, jnp.zeros((NPAD - N, x.shape[1]), jnp.float32)], axis=0)

    layers = [
        (Wq0, bq0, Wk0, bk0, Wv0, bv0, Ws0, bs0),
        (Wq1, bq1, Wk1, bk1, Wv1, bv1, Ws1, bs1),
        (Wq2, bq2, Wk2, bk2, Wv2, bv2, Ws2, bs2),
        (Wq3, bq3, Wk3, bk3, Wv3, bv3, Ws3, bs3),
    ]
    for (Wq, bq, Wk, bk, Wv, bv, Ws, bs) in layers:
        wall = jnp.concatenate([Wk, Wv, Wq, Ws], axis=1)
        ball = jnp.concatenate([bk, bv, bq, bs]).reshape(1, 4 * D_H)
        kv, q, s = _qkvs_matmul(h, wall, ball)
        hflat = _edge_kernel(q.reshape(-1), s.reshape(-1), kv,
                             src_sp, dst_sp, rowptr)
        h = hflat.reshape(NPAD, D_H)

    batch2d = jnp.concatenate(
        [batch, jnp.full((NPAD - N,), N_GRAPHS, jnp.int32)]).reshape(NPAD, 1)
    return _pool(h, batch2d, Wfin, bfin.reshape(1, 5))
